# baseline (speedup 1.0000x reference)
import jax
import jax.numpy as jnp
from jax import lax
from jax.experimental import pallas as pl
from jax.experimental.pallas import tpu as pltpu
from jax.experimental.pallas import tpu_sc as plsc

_CROP = 7
_C = 192
_CC = _C // 16
_N = 2000
_NPAD = 2048
_MROW = 80
_BOXW = 80
_OUT_TILE = _C * _CROP * _CROP


def _body(x0_hbm, y0_hbm, x1_hbm, y1_hbm, table_hbm, out_hbm,
          box_v, meta_i, meta_f,
          idx_a, idx_b, patch_a, patch_b, outt_a, outt_b, coord_i, coord_f,
          sg_a, sg_b, so_a, so_b):
    wid = lax.axis_index("c") * 16 + lax.axis_index("s")
    start = wid * 62 + jnp.minimum(wid, 16)
    cnt = jnp.where(wid < 16, 63, 62)
    astart = pl.multiple_of((start >> 3) << 3, 8)
    off_in = start - astart

    pltpu.sync_copy(x0_hbm.at[pl.ds(astart, _BOXW)], box_v.at[0])
    pltpu.sync_copy(y0_hbm.at[pl.ds(astart, _BOXW)], box_v.at[1])
    pltpu.sync_copy(x1_hbm.at[pl.ds(astart, _BOXW)], box_v.at[2])
    pltpu.sync_copy(y1_hbm.at[pl.ds(astart, _BOXW)], box_v.at[3])

    lane = lax.iota(jnp.int32, 16)
    lane_f = lane.astype(jnp.float32)

    one = jnp.full((16,), 1, jnp.int32)
    zero = jnp.full((16,), 0, jnp.int32)
    for q in range(4):
        sl = pl.ds(off_in + q * 16, 16)
        x0 = box_v[0, sl]
        y0 = box_v[1, sl]
        x1 = box_v[2, sl]
        y1 = box_v[3, sl]
        wh = (x1 - x0) * (y1 - y0)
        lev = (jnp.where(wh > 144.0, one, zero)
               + jnp.where(wh > 576.0, one, zero)
               + jnp.where(wh > 2304.0, one, zero))
        w_l = 128 >> lev
        off = jnp.where(lev == 0, 0,
                        jnp.where(lev == 1, 16384,
                                  jnp.where(lev == 2, 20480, 21504)))
        inv = jnp.where(lev == 0, 0.25,
                        jnp.where(lev == 1, 0.125,
                                  jnp.where(lev == 2, 0.0625, 0.03125)))
        bx0 = x0 * inv
        by0 = y0 * inv
        spanx = (x1 - x0) * inv
        spany = (y1 - y0) * inv
        t0 = jnp.float32(0.5 / 7.0)
        xb = jnp.clip((bx0 + spanx * t0).astype(jnp.int32), 0, w_l - 8)
        yb = jnp.clip((by0 + spany * t0).astype(jnp.int32), 0, w_l - 8)
        meta_i[pl.ds(0 * _MROW + q * 16, 16)] = off + yb * w_l + xb
        meta_i[pl.ds(1 * _MROW + q * 16, 16)] = w_l
        meta_i[pl.ds(2 * _MROW + q * 16, 16)] = xb
        meta_i[pl.ds(3 * _MROW + q * 16, 16)] = yb
        meta_f[pl.ds(0 * _MROW + q * 16, 16)] = bx0
        meta_f[pl.ds(1 * _MROW + q * 16, 16)] = by0
        meta_f[pl.ds(2 * _MROW + q * 16, 16)] = spanx
        meta_f[pl.ds(3 * _MROW + q * 16, 16)] = spany

    cvec_cc = [lane + cc * 16 for cc in range(_CC)]
    tvec = (lane_f + 0.5) / 7.0
    bufs = ((idx_a, patch_a, outt_a, sg_a, so_a),
            (idx_b, patch_b, outt_b, sg_b, so_b))

    def issue_gather(p, b):
        idx_r, patch_r, _, sg, _ = bufs[b]

        @pl.when(p < cnt)
        def _():
            base = meta_i[pl.ds(0 * _MROW + p, 16)][0]
            w_l = meta_i[pl.ds(1 * _MROW + p, 16)][0]
            for q in range(4):
                lin = lane + q * 16
                idx_r[pl.ds(q * 16, 16)] = base + (lin >> 3) * w_l + (lin & 7)
            pltpu.async_copy(table_hbm.at[idx_r], patch_r, sg)

    issue_gather(0, 0)

    def pair_body(k, _):
        for b in (0, 1):
            p = k * 2 + b
            idx_r, patch_r, outt_r, sg, so = bufs[b]

            @pl.when(p < cnt)
            def _():
                issue_gather(p + 1, 1 - b)

                w_l = meta_i[pl.ds(1 * _MROW + p, 16)][0]
                xb = meta_i[pl.ds(2 * _MROW + p, 16)][0]
                yb = meta_i[pl.ds(3 * _MROW + p, 16)][0]
                bx0 = meta_f[pl.ds(0 * _MROW + p, 16)][0]
                by0 = meta_f[pl.ds(1 * _MROW + p, 16)][0]
                spanx = meta_f[pl.ds(2 * _MROW + p, 16)][0]
                spany = meta_f[pl.ds(3 * _MROW + p, 16)][0]

                wm1 = w_l - 1
                xs = bx0 + spanx * tvec
                ys = by0 + spany * tvec
                x0i = xs.astype(jnp.int32)
                y0i = ys.astype(jnp.int32)
                wxv = xs - x0i.astype(jnp.float32)
                wyv = ys - y0i.astype(jnp.float32)
                x0c = jnp.minimum(x0i, wm1)
                y0c = jnp.minimum(y0i, wm1)
                x1c = jnp.minimum(x0i + 1, wm1)
                y1c = jnp.minimum(y0i + 1, wm1)
                rx0v = jnp.clip(x0c - xb, 0, 7)
                rx1v = jnp.clip(x1c - xb, 0, 7)
                coord_i[pl.ds(0, 16)] = jnp.clip(y0c - yb, 0, 7) * 8
                coord_i[pl.ds(16, 16)] = jnp.clip(y1c - yb, 0, 7) * 8
                coord_f[pl.ds(0, 16)] = wyv

                rx0s = [rx0v[j] for j in range(7)]
                rx1s = [rx1v[j] for j in range(7)]
                wxs = [wxv[j] for j in range(7)]

                pltpu.make_async_copy(table_hbm.at[idx_r], patch_r, sg).wait()

                @pl.when(p >= 2)
                def _():
                    pltpu.make_async_copy(outt_r, out_hbm.at[0], so).wait()

                def body_i(i, _):
                    ry0 = coord_i[pl.ds(i, 16)][0]
                    ry1 = coord_i[pl.ds(16 + i, 16)][0]
                    wyi = coord_f[pl.ds(i, 16)][0]
                    omy = 1.0 - wyi
                    i7 = i * 7
                    for j in range(7):
                        wxj = wxs[j]
                        omx = 1.0 - wxj
                        wv00 = jnp.broadcast_to(omy * omx, (16,))
                        wv01 = jnp.broadcast_to(omy * wxj, (16,))
                        wv10 = jnp.broadcast_to(wyi * omx, (16,))
                        wv11 = jnp.broadcast_to(wyi * wxj, (16,))
                        r00 = ry0 + rx0s[j]
                        r01 = ry0 + rx1s[j]
                        r10 = ry1 + rx0s[j]
                        r11 = ry1 + rx1s[j]
                        iv = jnp.full((16,), 0, jnp.int32) + i
                        jv = jnp.full((16,), j, jnp.int32)
                        vals = []
                        for cc in range(_CC):
                            sl = pl.ds(cc * 16, 16)
                            vals.append(
                                wv00 * patch_r[r00, sl]
                                + wv01 * patch_r[r01, sl]
                                + wv10 * patch_r[r10, sl]
                                + wv11 * patch_r[r11, sl])
                        for cc in range(_CC):
                            plsc.store_scatter(
                                outt_r, [cvec_cc[cc], iv, jv], vals[cc])
                    return 0

                lax.fori_loop(0, 7, body_i, 0)
                pltpu.async_copy(outt_r, out_hbm.at[start + p], so)

        return 0

    lax.fori_loop(0, 32, pair_body, 0)
    pltpu.make_async_copy(outt_a, out_hbm.at[0], so_a).wait()
    pltpu.make_async_copy(outt_b, out_hbm.at[0], so_b).wait()


def kernel(fs0, fs1, fs2, fs3, proposals):
    table = jnp.concatenate(
        [f[0].transpose(1, 2, 0).reshape(-1, _C) for f in (fs0, fs1, fs2, fs3)],
        axis=0)
    n = proposals.shape[0]
    boxes = proposals[:, 1:5]
    boxes = jnp.pad(boxes, ((0, _NPAD - n), (0, 0)))
    x0 = boxes[:, 0]
    y0 = boxes[:, 1]
    x1 = boxes[:, 2]
    y1 = boxes[:, 3]

    run = pl.kernel(
        _body,
        out_type=jax.ShapeDtypeStruct((_N, _C, _CROP, _CROP), jnp.float32),
        mesh=plsc.VectorSubcoreMesh(core_axis_name="c", subcore_axis_name="s"),
        compiler_params=pltpu.CompilerParams(
            use_tc_tiling_on_sc=False, needs_layout_passes=False),
        scratch_types=[
            pltpu.VMEM((4, _BOXW), jnp.float32),
            pltpu.VMEM((4 * _MROW,), jnp.int32),
            pltpu.VMEM((4 * _MROW,), jnp.float32),
            pltpu.VMEM((64,), jnp.int32),
            pltpu.VMEM((64,), jnp.int32),
            pltpu.VMEM((64, _C), jnp.float32),
            pltpu.VMEM((64, _C), jnp.float32),
            pltpu.VMEM((_C, _CROP, _CROP), jnp.float32),
            pltpu.VMEM((_C, _CROP, _CROP), jnp.float32),
            pltpu.VMEM((32,), jnp.int32),
            pltpu.VMEM((16,), jnp.float32),
            pltpu.SemaphoreType.DMA,
            pltpu.SemaphoreType.DMA,
            pltpu.SemaphoreType.DMA,
            pltpu.SemaphoreType.DMA,
        ],
    )
    return run(x0, y0, x1, y1, table)


# speedup vs baseline: 6.5759x; 6.5759x over previous
import jax
import jax.numpy as jnp
from jax import lax
from jax.experimental import pallas as pl
from jax.experimental.pallas import tpu as pltpu
from jax.experimental.pallas import tpu_sc as plsc

_CROP = 7
_C = 192
_CC = _C // 16
_N = 2000
_NPAD = 2048
_G = 128
_MROW = 144


def _body(x0_hbm, y0_hbm, x1_hbm, y1_hbm, table_hbm, out_hbm,
          box_v, meta_i, meta_f, idx_a, idx_b, rows_a, rows_b, wgt_v,
          agg_a, agg_b, sg_a, sg_b, so_a, so_b):
    tid = lax.axis_index("c") * 16 + lax.axis_index("s")
    g = tid & 15
    par = tid >> 4
    nunits = 25 - par

    base_n = g * _G
    pltpu.sync_copy(x0_hbm.at[pl.ds(base_n, _G)], box_v.at[0])
    pltpu.sync_copy(y0_hbm.at[pl.ds(base_n, _G)], box_v.at[1])
    pltpu.sync_copy(x1_hbm.at[pl.ds(base_n, _G)], box_v.at[2])
    pltpu.sync_copy(y1_hbm.at[pl.ds(base_n, _G)], box_v.at[3])

    lane = lax.iota(jnp.int32, 16)
    one = jnp.full((16,), 1, jnp.int32)
    zero = jnp.full((16,), 0, jnp.int32)

    for q in range(8):
        sl = pl.ds(q * 16, 16)
        x0 = box_v[0, sl]
        y0 = box_v[1, sl]
        x1 = box_v[2, sl]
        y1 = box_v[3, sl]
        wh = (x1 - x0) * (y1 - y0)
        lev = (jnp.where(wh > 144.0, one, zero)
               + jnp.where(wh > 576.0, one, zero)
               + jnp.where(wh > 2304.0, one, zero))
        w_l = 128 >> lev
        off = jnp.where(lev == 0, 0,
                        jnp.where(lev == 1, 16384,
                                  jnp.where(lev == 2, 20480, 21504)))
        inv = jnp.where(lev == 0, 0.25,
                        jnp.where(lev == 1, 0.125,
                                  jnp.where(lev == 2, 0.0625, 0.03125)))
        meta_f[pl.ds(0 * _MROW + q * 16, 16)] = x0 * inv
        meta_f[pl.ds(1 * _MROW + q * 16, 16)] = y0 * inv
        meta_f[pl.ds(2 * _MROW + q * 16, 16)] = (x1 - x0) * inv
        meta_f[pl.ds(3 * _MROW + q * 16, 16)] = (y1 - y0) * inv
        meta_i[pl.ds(0 * _MROW + q * 16, 16)] = w_l
        meta_i[pl.ds(1 * _MROW + q * 16, 16)] = off

    cb_cc = [(lane >> 3) + 2 * cc for cc in range(_CC)]
    clv = lane & 7

    wgt_v[pl.ds(4 * _MROW, 16)] = (lane.astype(jnp.float32) + 0.5) / 7.0

    idx_bufs = (idx_a, idx_b)
    rows_bufs = (rows_a, rows_b)
    sg = (sg_a, sg_b)
    agg_bufs = (agg_a, agg_b)
    so = (so_a, so_b)

    def gather(unit, b):
        buf = b & 1

        @pl.when(unit < nunits)
        def _():
            s = 2 * unit + par
            i = s // 7
            j = s - i * 7
            ti = wgt_v[pl.ds(4 * _MROW + i, 16)][0]
            tj = wgt_v[pl.ds(4 * _MROW + j, 16)][0]
            idx_r = idx_bufs[buf]
            for q in range(2):
                o = b * 32 + q * 16
                bx0 = meta_f[pl.ds(0 * _MROW + o, 16)]
                by0 = meta_f[pl.ds(1 * _MROW + o, 16)]
                spanx = meta_f[pl.ds(2 * _MROW + o, 16)]
                spany = meta_f[pl.ds(3 * _MROW + o, 16)]
                w_l = meta_i[pl.ds(0 * _MROW + o, 16)]
                off = meta_i[pl.ds(1 * _MROW + o, 16)]
                wm1 = w_l - 1
                xs = bx0 + spanx * tj
                ys = by0 + spany * ti
                x0i = xs.astype(jnp.int32)
                y0i = ys.astype(jnp.int32)
                wx = xs - x0i.astype(jnp.float32)
                wy = ys - y0i.astype(jnp.float32)
                x0c = jnp.minimum(x0i, wm1)
                y0c = jnp.minimum(y0i, wm1)
                x1c = jnp.minimum(x0i + 1, wm1)
                y1c = jnp.minimum(y0i + 1, wm1)
                r0 = off + y0c * w_l
                r1 = off + y1c * w_l
                idx_r[pl.ds(0 + q * 16, 16)] = r0 + x0c
                idx_r[pl.ds(32 + q * 16, 16)] = r0 + x1c
                idx_r[pl.ds(64 + q * 16, 16)] = r1 + x0c
                idx_r[pl.ds(96 + q * 16, 16)] = r1 + x1c
                omx = 1.0 - wx
                omy = 1.0 - wy
                wgt_v[pl.ds(0 * _MROW + o, 16)] = omy * omx
                wgt_v[pl.ds(1 * _MROW + o, 16)] = omy * wx
                wgt_v[pl.ds(2 * _MROW + o, 16)] = wy * omx
                wgt_v[pl.ds(3 * _MROW + o, 16)] = wy * wx
            pltpu.async_copy(table_hbm.at[idx_r], rows_bufs[buf], sg[buf])

    gather(0, 0)
    gather(0, 1)

    def pair_body(kp, _):
        for up in (0, 1):
            unit = 2 * kp + up

            @pl.when(unit < nunits)
            def _():
                s = 2 * unit + par
                i = s // 7
                j = s - i * 7
                agg_r = agg_bufs[up]

                for b in range(4):
                    buf = b & 1
                    rows_r = rows_bufs[buf]
                    pltpu.make_async_copy(
                        table_hbm.at[idx_bufs[buf]], rows_r, sg[buf]).wait()

                    if b == 0:
                        @pl.when(unit >= 2)
                        def _():
                            pltpu.make_async_copy(
                                agg_r, out_hbm.at[0, 0, :, 0], so[up]).wait()

                    b32 = b * 32

                    def k_body(k, _):
                        kk = b32 + k
                        w00 = wgt_v[pl.ds(0 * _MROW + kk, 16)][0]
                        w01 = wgt_v[pl.ds(1 * _MROW + kk, 16)][0]
                        w10 = wgt_v[pl.ds(2 * _MROW + kk, 16)][0]
                        w11 = wgt_v[pl.ds(3 * _MROW + kk, 16)][0]
                        vals = []
                        for cc in range(_CC):
                            sl = pl.ds(cc * 16, 16)
                            vals.append(w00 * rows_r[k, sl]
                                        + w01 * rows_r[32 + k, sl]
                                        + w10 * rows_r[64 + k, sl]
                                        + w11 * rows_r[96 + k, sl])
                        kv = jnp.full((16,), 0, jnp.int32) + kk
                        for cc in range(_CC):
                            plsc.store_scatter(
                                agg_r, [cb_cc[cc], clv, kv], vals[cc])
                        return 0

                    lax.fori_loop(0, 32, k_body, 0)

                    if b < 2:
                        gather(unit, b + 2)
                    else:
                        gather(unit + 1, b - 2)

                pltpu.async_copy(agg_r, out_hbm.at[i, j, :, g], so[up])

        return 0

    lax.fori_loop(0, 13, pair_body, 0)
    pltpu.make_async_copy(agg_a, out_hbm.at[0, 0, :, 0], so_a).wait()
    pltpu.make_async_copy(agg_b, out_hbm.at[0, 0, :, 0], so_b).wait()


def kernel(fs0, fs1, fs2, fs3, proposals):
    table = jnp.concatenate(
        [f[0].transpose(1, 2, 0).reshape(-1, _C) for f in (fs0, fs1, fs2, fs3)],
        axis=0)
    n = proposals.shape[0]
    boxes = proposals[:, 1:5]
    boxes = jnp.pad(boxes, ((0, _NPAD - n), (0, 0)))
    x0 = boxes[:, 0]
    y0 = boxes[:, 1]
    x1 = boxes[:, 2]
    y1 = boxes[:, 3]

    run = pl.kernel(
        _body,
        out_type=jax.ShapeDtypeStruct((_CROP, _CROP, 24, 16, 8, 128),
                                      jnp.float32),
        mesh=plsc.VectorSubcoreMesh(core_axis_name="c", subcore_axis_name="s"),
        compiler_params=pltpu.CompilerParams(
            use_tc_tiling_on_sc=False, needs_layout_passes=False),
        scratch_types=[
            pltpu.VMEM((4, _G), jnp.float32),
            pltpu.VMEM((2 * _MROW,), jnp.int32),
            pltpu.VMEM((4 * _MROW,), jnp.float32),
            pltpu.VMEM((128,), jnp.int32),
            pltpu.VMEM((128,), jnp.int32),
            pltpu.VMEM((128, _C), jnp.float32),
            pltpu.VMEM((128, _C), jnp.float32),
            pltpu.VMEM((4 * _MROW + 32,), jnp.float32),
            pltpu.VMEM((24, 8, 128), jnp.float32),
            pltpu.VMEM((24, 8, 128), jnp.float32),
            pltpu.SemaphoreType.DMA,
            pltpu.SemaphoreType.DMA,
            pltpu.SemaphoreType.DMA,
            pltpu.SemaphoreType.DMA,
        ],
    )
    out6 = run(x0, y0, x1, y1, table)
    r = out6.transpose(3, 5, 2, 4, 0, 1).reshape(_NPAD, _C, _CROP, _CROP)
    return r[:n]
